# baseline (speedup 1.0000x reference)
import functools

import jax
import jax.numpy as jnp
from jax import lax
from jax.experimental import pallas as pl
from jax.experimental.pallas import tpu as pltpu
from jax.experimental.pallas import tpu_sc as plsc

NUM_CLASSES = 1000000
EMBED_DIM = 32
BATCH = 16384
FIELDS = 26

_B = BATCH * FIELDS
_CW = 128
_NCHUNK = _B // _CW
_NW = 32
_CPW = _NCHUNK // _NW
_K = 4
_G = _CPW // _K
_GROWS = _K * _CW

_TCOLS = 16384
_TGRID = -(-NUM_CLASSES // _TCOLS)
_TROWS = _TGRID * _TCOLS


def _detile_body(in_ref, out_ref):
    x = in_ref[...]
    q = _TCOLS // 4
    eye = jnp.eye(EMBED_DIM, dtype=jnp.float32)
    parts = [
        jax.lax.dot_general(
            x[:, u * q:(u + 1) * q],
            eye,
            (((0,), (0,)), ((), ())),
            preferred_element_type=jnp.float32,
        )
        for u in range(4)
    ]
    out_ref[...] = jnp.concatenate(parts, axis=1)


def _detile(tt):
    return pl.pallas_call(
        _detile_body,
        grid=(_TGRID,),
        in_specs=[pl.BlockSpec((EMBED_DIM, _TCOLS), lambda j: (0, j))],
        out_specs=pl.BlockSpec((_TCOLS // 4, 128), lambda j: (j, 0)),
        out_shape=jax.ShapeDtypeStruct((_TROWS // 4, 128), jnp.float32),
    )(tt)


def _make_gather():
    mesh = plsc.VectorSubcoreMesh(core_axis_name="c", subcore_axis_name="s")

    @functools.partial(
        pl.kernel,
        mesh=mesh,
        compiler_params=pltpu.CompilerParams(use_tc_tiling_on_sc=False),
        out_type=jax.ShapeDtypeStruct((_B, EMBED_DIM), jnp.float32),
        scratch_types=[
            pltpu.VMEM((_CPW, _CW), jnp.int32),
            pltpu.VMEM((_GROWS, EMBED_DIM), jnp.float32),
            pltpu.VMEM((_GROWS, EMBED_DIM), jnp.float32),
            pltpu.SemaphoreType.DMA,
            pltpu.SemaphoreType.DMA,
        ],
    )
    def gather_kernel(idx_hbm, table_hbm, out_hbm, idx_v, buf0, buf1, sem0, sem1):
        nc = 2
        wid = lax.axis_index("s") * nc + lax.axis_index("c")
        base = wid * _CPW
        pltpu.sync_copy(idx_hbm.at[pl.ds(base, _CPW)], idx_v)

        bufs = (buf0, buf1)
        sems = (sem0, sem1)

        def fire(g, b):
            for t in range(_K):
                pltpu.async_copy(
                    table_hbm.at[idx_v.at[g * _K + t]],
                    bufs[b].at[pl.ds(t * _CW, _CW)],
                    sems[b],
                )

        def drain(b):
            pltpu.make_async_copy(
                table_hbm.at[pl.ds(0, _GROWS)], bufs[b], sems[b]
            ).wait()

        def writeback(g, b):
            pltpu.sync_copy(
                bufs[b], out_hbm.at[pl.ds((base + g * _K) * _CW, _GROWS)]
            )

        fire(0, 0)
        fire(1, 1)

        def outer(og, carry):
            for b in range(2):
                g = og * 2 + b
                drain(b)
                writeback(g, b)
                fire(g + 2, b)
            return carry

        lax.fori_loop(0, _G // 2 - 1, outer, 0)

        for b in range(2):
            drain(b)
            writeback(_G - 2 + b, b)

    return gather_kernel


_gather = _make_gather()


@jax.jit
def kernel(x, embed_map):
    xf = x.T.reshape(_NCHUNK, _CW).astype(jnp.int32)
    q = _TCOLS // 4
    xf = (
        (xf & ~(_TCOLS - 1))
        | ((xf & (q - 1)) << 2)
        | ((xf // q) & 3)
    )
    t_lin = _detile(embed_map.T).reshape(_TROWS, EMBED_DIM)
    out = _gather(xf, t_lin)
    return out.reshape(FIELDS, BATCH, EMBED_DIM).transpose(1, 0, 2)


# speedup vs baseline: 1.0060x; 1.0060x over previous
import functools

import jax
import jax.numpy as jnp
from jax import lax
from jax.experimental import pallas as pl
from jax.experimental.pallas import tpu as pltpu
from jax.experimental.pallas import tpu_sc as plsc

NUM_CLASSES = 1000000
EMBED_DIM = 32
BATCH = 16384
FIELDS = 26

_B = BATCH * FIELDS
_CW = 128
_NCHUNK = _B // _CW
_NW = 32
_CPW = _NCHUNK // _NW
_K = 4
_G = _CPW // _K
_GROWS = _K * _CW

_TCOLS = 32768
_TGRID = -(-NUM_CLASSES // _TCOLS)
_TROWS = _TGRID * _TCOLS


def _detile_body(in_ref, out_ref):
    x = in_ref[...]
    q = _TCOLS // 4
    parts = [jnp.transpose(x[:, u * q:(u + 1) * q]) for u in range(4)]
    out_ref[...] = jnp.concatenate(parts, axis=1)


def _detile(tt):
    return pl.pallas_call(
        _detile_body,
        grid=(_TGRID,),
        in_specs=[pl.BlockSpec((EMBED_DIM, _TCOLS), lambda j: (0, j))],
        out_specs=pl.BlockSpec((_TCOLS // 4, 128), lambda j: (j, 0)),
        out_shape=jax.ShapeDtypeStruct((_TROWS // 4, 128), jnp.float32),
    )(tt)


def _make_gather():
    mesh = plsc.VectorSubcoreMesh(core_axis_name="c", subcore_axis_name="s")

    @functools.partial(
        pl.kernel,
        mesh=mesh,
        compiler_params=pltpu.CompilerParams(use_tc_tiling_on_sc=False),
        out_type=jax.ShapeDtypeStruct((_B, EMBED_DIM), jnp.float32),
        scratch_types=[
            pltpu.VMEM((_CPW, _CW), jnp.int32),
            pltpu.VMEM((_GROWS, EMBED_DIM), jnp.float32),
            pltpu.VMEM((_GROWS, EMBED_DIM), jnp.float32),
            pltpu.SemaphoreType.DMA,
            pltpu.SemaphoreType.DMA,
        ],
    )
    def gather_kernel(idx_hbm, table_hbm, out_hbm, idx_v, buf0, buf1, sem0, sem1):
        nc = 2
        wid = lax.axis_index("s") * nc + lax.axis_index("c")
        base = wid * _CPW
        pltpu.sync_copy(idx_hbm.at[pl.ds(base, _CPW)], idx_v)

        bufs = (buf0, buf1)
        sems = (sem0, sem1)

        def fire(g, b):
            for t in range(_K):
                pltpu.async_copy(
                    table_hbm.at[idx_v.at[g * _K + t]],
                    bufs[b].at[pl.ds(t * _CW, _CW)],
                    sems[b],
                )

        def drain(b):
            pltpu.make_async_copy(
                table_hbm.at[pl.ds(0, _GROWS)], bufs[b], sems[b]
            ).wait()

        def writeback(g, b):
            pltpu.sync_copy(
                bufs[b], out_hbm.at[pl.ds((base + g * _K) * _CW, _GROWS)]
            )

        fire(0, 0)
        fire(1, 1)

        def outer(og, carry):
            for b in range(2):
                g = og * 2 + b
                drain(b)
                writeback(g, b)
                fire(g + 2, b)
            return carry

        lax.fori_loop(0, _G // 2 - 1, outer, 0)

        for b in range(2):
            drain(b)
            writeback(_G - 2 + b, b)

    return gather_kernel


_gather = _make_gather()


@jax.jit
def kernel(x, embed_map):
    xf = x.T.reshape(_NCHUNK, _CW).astype(jnp.int32)
    q = _TCOLS // 4
    xf = (
        (xf & ~(_TCOLS - 1))
        | ((xf & (q - 1)) << 2)
        | ((xf // q) & 3)
    )
    t_lin = _detile(embed_map.T).reshape(_TROWS, EMBED_DIM)
    out = _gather(xf, t_lin)
    return out.reshape(FIELDS, BATCH, EMBED_DIM).transpose(1, 0, 2)
